# baseline (speedup 1.0000x reference)
import functools

import jax
import jax.numpy as jnp
from jax import lax
from jax.experimental import pallas as pl
from jax.experimental.pallas import tpu as pltpu
from jax.experimental.pallas import tpu_sc as plsc

E = 8
TOP_K = 2
H = 1024
I = 1024
N = 2048
TMS = 128
P = N * TOP_K + E * TMS
G = P // TMS
TE_LANES = 128

TMR = 512
NTR = N // TMR

_NC = 2
_NS = 16
_NW = _NC * _NS
_TPW = N // _NW


def _router_body(x_ref, gw_ref, alpha_ref, dest8_ref, w8_ref, te_ref,
                 xb_ref, cnt_ref, run_ref):
    p = pl.program_id(0)
    t = pl.program_id(1)
    x = x_ref[...]
    xb_ref[...] = x.astype(jnp.bfloat16)
    logits = jnp.dot(x, gw_ref[...], preferred_element_type=jnp.float32)
    probs = jax.nn.softmax(logits, axis=-1)
    m1 = jnp.max(probs, axis=-1, keepdims=True)
    masked = jnp.where(probs >= m1, -1.0, probs)
    m2 = jnp.max(masked, axis=-1, keepdims=True)
    sel = (probs >= m2).astype(jnp.float32)
    colsum = jnp.sum(sel, axis=0, keepdims=True)

    @pl.when(p == 0)
    def _():
        prev = jnp.where(t == 0, jnp.zeros_like(colsum), cnt_ref[...])
        cnt_ref[...] = prev + colsum

    @pl.when(p == 1)
    def _():
        cnt = cnt_ref[...]
        padded = jnp.floor((cnt + (TMS - 1)) * (1.0 / TMS)) * TMS
        triu = (jax.lax.broadcasted_iota(jnp.int32, (E, E), 0)
                <= jax.lax.broadcasted_iota(jnp.int32, (E, E), 1)
                ).astype(jnp.float32)
        cum = jnp.dot(padded, triu, preferred_element_type=jnp.float32)
        offs = cum - padded

        run = jnp.where(t == 0, jnp.zeros_like(colsum), run_ref[...])
        run_ref[...] = run + colsum
        tril = (jax.lax.broadcasted_iota(jnp.int32, (TMR, TMR), 1)
                < jax.lax.broadcasted_iota(jnp.int32, (TMR, TMR), 0)
                ).astype(jnp.float32)
        ranks = jnp.dot(tril, sel, preferred_element_type=jnp.float32)
        dest_all = offs + run + ranks

        lane = jax.lax.broadcasted_iota(jnp.int32, probs.shape, 1)
        i1 = jnp.min(jnp.where(probs >= m1, lane, E), axis=-1, keepdims=True)
        i2 = jnp.min(jnp.where((probs >= m2) & (lane != i1), lane, E),
                     axis=-1, keepdims=True)
        hit1 = lane == i1
        hit2 = lane == i2
        d0 = jnp.sum(jnp.where(hit1, dest_all, 0.0), axis=-1)
        d1 = jnp.sum(jnp.where(hit2, dest_all, 0.0), axis=-1)
        wa = probs * alpha_ref[...]
        w0 = jnp.sum(jnp.where(hit1, wa, 0.0), axis=-1)
        w1 = jnp.sum(jnp.where(hit2, wa, 0.0), axis=-1)

        su = jax.lax.broadcasted_iota(jnp.int32, (8, TMR), 0)
        dest8_ref[...] = jnp.where(
            su == 0, d0[None, :], jnp.where(su == 1, d1[None, :], 0.0)
        ).astype(jnp.int32)
        w8_ref[...] = jnp.where(
            su == 0, w0[None, :], jnp.where(su == 1, w1[None, :], 0.0))

        gt = (jax.lax.broadcasted_iota(jnp.int32, (1, TE_LANES), 1)
              * TMS).astype(jnp.float32)
        te = jnp.zeros((1, TE_LANES), jnp.float32)
        for e in range(E):
            te = te + (gt >= cum[0, e]).astype(jnp.float32)
        te_ref[...] = jnp.minimum(te, E - 1).astype(jnp.int32)


def _router(flat, gate_w, alpha_row):
    return pl.pallas_call(
        _router_body,
        grid=(2, NTR),
        in_specs=[
            pl.BlockSpec((TMR, H), lambda p, t: (t, 0)),
            pl.BlockSpec((H, E), lambda p, t: (0, 0)),
            pl.BlockSpec((1, E), lambda p, t: (0, 0)),
        ],
        out_specs=[
            pl.BlockSpec((8, TMR), lambda p, t: (0, t)),
            pl.BlockSpec((8, TMR), lambda p, t: (0, t)),
            pl.BlockSpec((1, TE_LANES), lambda p, t: (0, 0)),
            pl.BlockSpec((TMR, H), lambda p, t: (t, 0)),
        ],
        out_shape=[
            jax.ShapeDtypeStruct((8, N), jnp.int32),
            jax.ShapeDtypeStruct((8, N), jnp.float32),
            jax.ShapeDtypeStruct((1, TE_LANES), jnp.int32),
            jax.ShapeDtypeStruct((N, H), jnp.bfloat16),
        ],
        scratch_shapes=[
            pltpu.VMEM((1, E), jnp.float32),
            pltpu.VMEM((1, E), jnp.float32),
        ],
    )(flat, gate_w, alpha_row)


@functools.lru_cache(maxsize=None)
def _make_sc_dispatch():
    mesh = plsc.VectorSubcoreMesh(core_axis_name="c", subcore_axis_name="s",
                                  num_cores=_NC)

    @functools.partial(
        pl.kernel,
        mesh=mesh,
        out_type=jax.ShapeDtypeStruct((P, H // 2), jnp.int32),
        scratch_types=[
            pltpu.VMEM((_TPW, H // 2), jnp.int32),
            pltpu.VMEM((_TPW,), jnp.int32),
            pltpu.VMEM((_TPW,), jnp.int32),
            pltpu.SemaphoreType.DMA,
            pltpu.SemaphoreType.DMA,
        ],
    )
    def dispatch_k(x_hbm, dest8_hbm, xg_hbm, rows_v, i0_v, i1_v, s0, s1):
        wid = lax.axis_index("s") * _NC + lax.axis_index("c")
        base = wid * _TPW
        pltpu.sync_copy(dest8_hbm.at[0, pl.ds(base, _TPW)], i0_v)
        pltpu.sync_copy(dest8_hbm.at[1, pl.ds(base, _TPW)], i1_v)
        pltpu.sync_copy(x_hbm.at[pl.ds(base, _TPW)], rows_v)
        c0 = pltpu.async_copy(rows_v, xg_hbm.at[i0_v], s0)
        c1 = pltpu.async_copy(rows_v, xg_hbm.at[i1_v], s1)
        c0.wait()
        c1.wait()

    return dispatch_k


@functools.lru_cache(maxsize=None)
def _make_sc_combine():
    mesh = plsc.VectorSubcoreMesh(core_axis_name="c", subcore_axis_name="s",
                                  num_cores=_NC)

    @functools.partial(
        pl.kernel,
        mesh=mesh,
        out_type=jax.ShapeDtypeStruct((TOP_K * N, H // 2), jnp.int32),
        scratch_types=[
            pltpu.VMEM((_TPW, H // 2), jnp.int32),
            pltpu.VMEM((_TPW,), jnp.int32),
            pltpu.SemaphoreType.DMA,
        ],
    )
    def combine_k(ys_hbm, dest8_hbm, g2_hbm, rows_v, idx_v, sem):
        wid = lax.axis_index("s") * _NC + lax.axis_index("c")
        base = wid * _TPW
        for slot in range(TOP_K):
            pltpu.sync_copy(dest8_hbm.at[slot, pl.ds(base, _TPW)], idx_v)
            pltpu.async_copy(ys_hbm.at[idx_v], rows_v, sem).wait()
            pltpu.sync_copy(rows_v, g2_hbm.at[pl.ds(slot * N + base, _TPW)])

    return combine_k


def _ffn_body(te_ref, xg_ref, f1w_ref, f1b_ref, f2w_ref, f2b_ref, ys_ref):
    xb = xg_ref[...]
    h1 = jnp.dot(xb, f1w_ref[0], preferred_element_type=jnp.float32)
    h1 = h1 + f1b_ref[0, 0, :][None, :]
    g = 0.5 * h1 * (1.0 + jax.lax.erf(h1 * 0.7071067811865476))
    y = jnp.dot(g.astype(jnp.bfloat16), f2w_ref[0],
                preferred_element_type=jnp.float32)
    ys_ref[...] = (y + f2b_ref[0, 0, :][None, :]).astype(jnp.bfloat16)


def _ffn(xg, f1w, f1b, f2w, f2b, tile_expert):
    grid_spec = pltpu.PrefetchScalarGridSpec(
        num_scalar_prefetch=1,
        grid=(G,),
        in_specs=[
            pl.BlockSpec((TMS, H), lambda g, te: (g, 0)),
            pl.BlockSpec((1, H, I), lambda g, te: (te[g], 0, 0)),
            pl.BlockSpec((1, 1, I), lambda g, te: (te[g], 0, 0)),
            pl.BlockSpec((1, I, H), lambda g, te: (te[g], 0, 0)),
            pl.BlockSpec((1, 1, H), lambda g, te: (te[g], 0, 0)),
        ],
        out_specs=pl.BlockSpec((TMS, H), lambda g, te: (g, 0)),
    )
    return pl.pallas_call(
        _ffn_body,
        grid_spec=grid_spec,
        out_shape=jax.ShapeDtypeStruct((P, H), jnp.bfloat16),
    )(tile_expert, xg, f1w, f1b, f2w, f2b)


def _wadd_body(g_ref, w_ref, out_ref):
    w0 = w_ref[0, :][:, None]
    w1 = w_ref[1, :][:, None]
    out_ref[...] = (g_ref[0].astype(jnp.float32) * w0
                    + g_ref[1].astype(jnp.float32) * w1)


def _combine_add(g2, w8):
    tm = 512
    return pl.pallas_call(
        _wadd_body,
        grid=(N // tm,),
        in_specs=[
            pl.BlockSpec((2, tm, H), lambda t: (0, t, 0)),
            pl.BlockSpec((8, tm), lambda t: (0, t)),
        ],
        out_specs=pl.BlockSpec((tm, H), lambda t: (t, 0)),
        out_shape=jax.ShapeDtypeStruct((N, H), jnp.float32),
    )(g2, w8)


def _sc_dispatch(x, dest8):
    return _make_sc_dispatch()(x, dest8)


def _sc_combine(ys, dest8):
    return _make_sc_combine()(ys, dest8)


def _as_i32(x16):
    n, h = x16.shape
    return jax.lax.bitcast_convert_type(
        x16.reshape(n, h // 2, 2), jnp.int32)


def _as_bf16(xi32):
    n, h2 = xi32.shape
    return jax.lax.bitcast_convert_type(xi32, jnp.bfloat16).reshape(n, h2 * 2)


@jax.jit
def _moe(flat, gate_w, alpha_row, f1w, f1b, f2w, f2b):
    dest8, w8, te, xb = _router(flat, gate_w, alpha_row)
    tile_expert = te[0, :G]
    xgi = _sc_dispatch(_as_i32(xb), dest8)
    ys = _ffn(_as_bf16(xgi), f1w, f1b, f2w, f2b, tile_expert)
    g2 = _sc_combine(_as_i32(ys), dest8)
    return _combine_add(_as_bf16(g2).reshape(TOP_K, N, H), w8)


def kernel(hidden_states, gate_w, fc1_w, fc1_b, fc2_w, fc2_b, alpha):
    b, s, h = hidden_states.shape
    flat = hidden_states.reshape(-1, h)
    f1w = fc1_w.astype(jnp.bfloat16)
    f2w = fc2_w.astype(jnp.bfloat16)
    f1b = fc1_b.reshape(E, 1, I)
    f2b = fc2_b.reshape(E, 1, H)
    out = _moe(flat, gate_w, alpha.reshape(1, E), f1w, f1b, f2w, f2b)
    return out.reshape(b, s, h)


# speedup vs baseline: 3.3063x; 3.3063x over previous
import functools

import jax
import jax.numpy as jnp
from jax import lax
from jax.experimental import pallas as pl
from jax.experimental.pallas import tpu as pltpu
from jax.experimental.pallas import tpu_sc as plsc

E = 8
TOP_K = 2
H = 1024
I = 1024
N = 2048
TMS = 128
P = N * TOP_K + E * TMS
G = P // TMS
TE_LANES = 128

TMR = 512
NTR = N // TMR

_NC = 2
_NS = 16
_NW = _NC * _NS
_TPW = N // _NW


def _router_body(x_ref, gw_ref, alpha_ref, dest8_ref, w8_ref, te_ref,
                 cnt_ref, run_ref):
    p = pl.program_id(0)
    t = pl.program_id(1)
    x = x_ref[...]
    logits = jnp.dot(x, gw_ref[...], preferred_element_type=jnp.float32)
    probs = jax.nn.softmax(logits, axis=-1)
    m1 = jnp.max(probs, axis=-1, keepdims=True)
    masked = jnp.where(probs >= m1, -1.0, probs)
    m2 = jnp.max(masked, axis=-1, keepdims=True)
    sel = (probs >= m2).astype(jnp.float32)
    colsum = jnp.sum(sel, axis=0, keepdims=True)

    @pl.when(p == 0)
    def _():
        prev = jnp.where(t == 0, jnp.zeros_like(colsum), cnt_ref[...])
        cnt_ref[...] = prev + colsum

    @pl.when(p == 1)
    def _():
        cnt = cnt_ref[...]
        padded = jnp.floor((cnt + (TMS - 1)) * (1.0 / TMS)) * TMS
        triu = (jax.lax.broadcasted_iota(jnp.int32, (E, E), 0)
                <= jax.lax.broadcasted_iota(jnp.int32, (E, E), 1)
                ).astype(jnp.float32)
        cum = jnp.dot(padded, triu, preferred_element_type=jnp.float32)
        offs = cum - padded

        run = jnp.where(t == 0, jnp.zeros_like(colsum), run_ref[...])
        run_ref[...] = run + colsum
        tril = (jax.lax.broadcasted_iota(jnp.int32, (TMR, TMR), 1)
                < jax.lax.broadcasted_iota(jnp.int32, (TMR, TMR), 0)
                ).astype(jnp.float32)
        ranks = jnp.dot(tril, sel, preferred_element_type=jnp.float32)
        dest_all = offs + run + ranks

        lane = jax.lax.broadcasted_iota(jnp.int32, probs.shape, 1)
        i1 = jnp.min(jnp.where(probs >= m1, lane, E), axis=-1, keepdims=True)
        i2 = jnp.min(jnp.where((probs >= m2) & (lane != i1), lane, E),
                     axis=-1, keepdims=True)
        hit1 = lane == i1
        hit2 = lane == i2
        d0 = jnp.sum(jnp.where(hit1, dest_all, 0.0), axis=-1)
        d1 = jnp.sum(jnp.where(hit2, dest_all, 0.0), axis=-1)
        wa = probs * alpha_ref[...]
        w0 = jnp.sum(jnp.where(hit1, wa, 0.0), axis=-1)
        w1 = jnp.sum(jnp.where(hit2, wa, 0.0), axis=-1)

        su = jax.lax.broadcasted_iota(jnp.int32, (8, TMR), 0)
        dest8_ref[...] = jnp.where(
            su == 0, d0[None, :], jnp.where(su == 1, d1[None, :], 0.0)
        ).astype(jnp.int32)
        w8_ref[...] = jnp.where(
            su == 0, w0[None, :], jnp.where(su == 1, w1[None, :], 0.0))

        gt = (jax.lax.broadcasted_iota(jnp.int32, (1, TE_LANES), 1)
              * TMS).astype(jnp.float32)
        te = jnp.zeros((1, TE_LANES), jnp.float32)
        for e in range(E):
            te = te + (gt >= cum[0, e]).astype(jnp.float32)
        te_ref[...] = jnp.minimum(te, E - 1).astype(jnp.int32)


def _router(flat, gate_w, alpha_row):
    return pl.pallas_call(
        _router_body,
        grid=(2, NTR),
        in_specs=[
            pl.BlockSpec((TMR, H), lambda p, t: (t, 0)),
            pl.BlockSpec((H, E), lambda p, t: (0, 0)),
            pl.BlockSpec((1, E), lambda p, t: (0, 0)),
        ],
        out_specs=[
            pl.BlockSpec((8, TMR), lambda p, t: (0, t)),
            pl.BlockSpec((8, TMR), lambda p, t: (0, t)),
            pl.BlockSpec((1, TE_LANES), lambda p, t: (0, 0)),
        ],
        out_shape=[
            jax.ShapeDtypeStruct((8, N), jnp.int32),
            jax.ShapeDtypeStruct((8, N), jnp.float32),
            jax.ShapeDtypeStruct((1, TE_LANES), jnp.int32),
        ],
        scratch_shapes=[
            pltpu.VMEM((1, E), jnp.float32),
            pltpu.VMEM((1, E), jnp.float32),
        ],
    )(flat, gate_w, alpha_row)


@functools.lru_cache(maxsize=None)
def _make_sc_dispatch():
    mesh = plsc.VectorSubcoreMesh(core_axis_name="c", subcore_axis_name="s",
                                  num_cores=_NC)

    @functools.partial(
        pl.kernel,
        mesh=mesh,
        out_type=jax.ShapeDtypeStruct((P, H), jnp.float32),
        scratch_types=[
            pltpu.VMEM((_TPW, H), jnp.float32),
            pltpu.VMEM((_TPW,), jnp.int32),
            pltpu.VMEM((_TPW,), jnp.int32),
            pltpu.SemaphoreType.DMA,
            pltpu.SemaphoreType.DMA,
        ],
    )
    def dispatch_k(x_hbm, dest8_hbm, xg_hbm, rows_v, i0_v, i1_v, s0, s1):
        wid = lax.axis_index("s") * _NC + lax.axis_index("c")
        base = wid * _TPW
        pltpu.sync_copy(dest8_hbm.at[0, pl.ds(base, _TPW)], i0_v)
        pltpu.sync_copy(dest8_hbm.at[1, pl.ds(base, _TPW)], i1_v)
        pltpu.sync_copy(x_hbm.at[pl.ds(base, _TPW)], rows_v)
        c0 = pltpu.async_copy(rows_v, xg_hbm.at[i0_v], s0)
        c1 = pltpu.async_copy(rows_v, xg_hbm.at[i1_v], s1)
        c0.wait()
        c1.wait()

    return dispatch_k


@functools.lru_cache(maxsize=None)
def _make_sc_combine():
    mesh = plsc.VectorSubcoreMesh(core_axis_name="c", subcore_axis_name="s",
                                  num_cores=_NC)

    @functools.partial(
        pl.kernel,
        mesh=mesh,
        out_type=jax.ShapeDtypeStruct((TOP_K * N, H), jnp.float32),
        scratch_types=[
            pltpu.VMEM((_TPW, H), jnp.float32),
            pltpu.VMEM((_TPW,), jnp.int32),
            pltpu.SemaphoreType.DMA,
        ],
    )
    def combine_k(ys_hbm, dest8_hbm, g2_hbm, rows_v, idx_v, sem):
        wid = lax.axis_index("s") * _NC + lax.axis_index("c")
        base = wid * _TPW
        for slot in range(TOP_K):
            pltpu.sync_copy(dest8_hbm.at[slot, pl.ds(base, _TPW)], idx_v)
            pltpu.async_copy(ys_hbm.at[idx_v], rows_v, sem).wait()
            pltpu.sync_copy(rows_v, g2_hbm.at[pl.ds(slot * N + base, _TPW)])

    return combine_k


def _ffn_body(te_ref, xg_ref, f1w_ref, f1b_ref, f2w_ref, f2b_ref, ys_ref):
    xb = xg_ref[...].astype(jnp.bfloat16)
    h1 = jnp.dot(xb, f1w_ref[0], preferred_element_type=jnp.float32)
    h1 = h1 + f1b_ref[0, 0, :][None, :]
    g = 0.5 * h1 * (1.0 + jax.lax.erf(h1 * 0.7071067811865476))
    y = jnp.dot(g.astype(jnp.bfloat16), f2w_ref[0],
                preferred_element_type=jnp.float32)
    ys_ref[...] = y + f2b_ref[0, 0, :][None, :]


def _ffn(xg, f1w, f1b, f2w, f2b, tile_expert):
    grid_spec = pltpu.PrefetchScalarGridSpec(
        num_scalar_prefetch=1,
        grid=(G,),
        in_specs=[
            pl.BlockSpec((TMS, H), lambda g, te: (g, 0)),
            pl.BlockSpec((1, H, I), lambda g, te: (te[g], 0, 0)),
            pl.BlockSpec((1, 1, I), lambda g, te: (te[g], 0, 0)),
            pl.BlockSpec((1, I, H), lambda g, te: (te[g], 0, 0)),
            pl.BlockSpec((1, 1, H), lambda g, te: (te[g], 0, 0)),
        ],
        out_specs=pl.BlockSpec((TMS, H), lambda g, te: (g, 0)),
    )
    return pl.pallas_call(
        _ffn_body,
        grid_spec=grid_spec,
        out_shape=jax.ShapeDtypeStruct((P, H), jnp.float32),
    )(tile_expert, xg, f1w, f1b, f2w, f2b)


def _wadd_body(g_ref, w_ref, out_ref):
    w0 = w_ref[0, :][:, None]
    w1 = w_ref[1, :][:, None]
    out_ref[...] = g_ref[0] * w0 + g_ref[1] * w1


def _combine_add(g2, w8):
    tm = 512
    return pl.pallas_call(
        _wadd_body,
        grid=(N // tm,),
        in_specs=[
            pl.BlockSpec((2, tm, H), lambda t: (0, t, 0)),
            pl.BlockSpec((8, tm), lambda t: (0, t)),
        ],
        out_specs=pl.BlockSpec((tm, H), lambda t: (t, 0)),
        out_shape=jax.ShapeDtypeStruct((N, H), jnp.float32),
    )(g2, w8)


def _sc_dispatch(x, dest8):
    return _make_sc_dispatch()(x, dest8)


def _sc_combine(ys, dest8):
    return _make_sc_combine()(ys, dest8)


@jax.jit
def _moe(flat, gate_w, alpha_row, f1w, f1b, f2w, f2b):
    dest8, w8, te = _router(flat, gate_w, alpha_row)
    tile_expert = te[0, :G]
    xg = _sc_dispatch(flat, dest8)
    ys = _ffn(xg, f1w, f1b, f2w, f2b, tile_expert)
    g2 = _sc_combine(ys, dest8)
    return _combine_add(g2.reshape(TOP_K, N, H), w8)


def kernel(hidden_states, gate_w, fc1_w, fc1_b, fc2_w, fc2_b, alpha):
    b, s, h = hidden_states.shape
    flat = hidden_states.reshape(-1, h)
    f1w = fc1_w.astype(jnp.bfloat16)
    f2w = fc2_w.astype(jnp.bfloat16)
    f1b = fc1_b.reshape(E, 1, I)
    f2b = fc2_b.reshape(E, 1, H)
    out = _moe(flat, gate_w, alpha.reshape(1, E), f1w, f1b, f2w, f2b)
    return out.reshape(b, s, h)


# speedup vs baseline: 3.4593x; 1.0463x over previous
import functools

import jax
import jax.numpy as jnp
from jax import lax
from jax.experimental import pallas as pl
from jax.experimental.pallas import tpu as pltpu
from jax.experimental.pallas import tpu_sc as plsc

E = 8
TOP_K = 2
H = 1024
I = 1024
N = 2048
TMS = 256
P = N * TOP_K + E * TMS
G = P // TMS
TE_LANES = 128

TMR = 512
NTR = N // TMR

_NC = 2
_NS = 16
_NW = _NC * _NS
_TPW = N // _NW


def _router_body(x_ref, gw_ref, alpha_ref, dest8_ref, w8_ref, te_ref,
                 cnt_ref, run_ref):
    p = pl.program_id(0)
    t = pl.program_id(1)
    x = x_ref[...]
    logits = jnp.dot(x, gw_ref[...], preferred_element_type=jnp.float32)
    probs = jax.nn.softmax(logits, axis=-1)
    m1 = jnp.max(probs, axis=-1, keepdims=True)
    masked = jnp.where(probs >= m1, -1.0, probs)
    m2 = jnp.max(masked, axis=-1, keepdims=True)
    sel = (probs >= m2).astype(jnp.float32)
    colsum = jnp.sum(sel, axis=0, keepdims=True)

    @pl.when(p == 0)
    def _():
        prev = jnp.where(t == 0, jnp.zeros_like(colsum), cnt_ref[...])
        cnt_ref[...] = prev + colsum

    @pl.when(p == 1)
    def _():
        cnt = cnt_ref[...]
        padded = jnp.floor((cnt + (TMS - 1)) * (1.0 / TMS)) * TMS
        triu = (jax.lax.broadcasted_iota(jnp.int32, (E, E), 0)
                <= jax.lax.broadcasted_iota(jnp.int32, (E, E), 1)
                ).astype(jnp.float32)
        cum = jnp.dot(padded, triu, preferred_element_type=jnp.float32)
        offs = cum - padded

        run = jnp.where(t == 0, jnp.zeros_like(colsum), run_ref[...])
        run_ref[...] = run + colsum
        tril = (jax.lax.broadcasted_iota(jnp.int32, (TMR, TMR), 1)
                < jax.lax.broadcasted_iota(jnp.int32, (TMR, TMR), 0)
                ).astype(jnp.float32)
        ranks = jnp.dot(tril, sel, preferred_element_type=jnp.float32)
        dest_all = offs + run + ranks

        lane = jax.lax.broadcasted_iota(jnp.int32, probs.shape, 1)
        i1 = jnp.min(jnp.where(probs >= m1, lane, E), axis=-1, keepdims=True)
        i2 = jnp.min(jnp.where((probs >= m2) & (lane != i1), lane, E),
                     axis=-1, keepdims=True)
        hit1 = lane == i1
        hit2 = lane == i2
        d0 = jnp.sum(jnp.where(hit1, dest_all, 0.0), axis=-1)
        d1 = jnp.sum(jnp.where(hit2, dest_all, 0.0), axis=-1)
        wa = probs * alpha_ref[...]
        w0 = jnp.sum(jnp.where(hit1, wa, 0.0), axis=-1)
        w1 = jnp.sum(jnp.where(hit2, wa, 0.0), axis=-1)

        su = jax.lax.broadcasted_iota(jnp.int32, (8, TMR), 0)
        dest8_ref[...] = jnp.where(
            su == 0, d0[None, :], jnp.where(su == 1, d1[None, :], 0.0)
        ).astype(jnp.int32)
        w8_ref[...] = jnp.where(
            su == 0, w0[None, :], jnp.where(su == 1, w1[None, :], 0.0))

        gt = (jax.lax.broadcasted_iota(jnp.int32, (1, TE_LANES), 1)
              * TMS).astype(jnp.float32)
        te = jnp.zeros((1, TE_LANES), jnp.float32)
        for e in range(E):
            te = te + (gt >= cum[0, e]).astype(jnp.float32)
        te_ref[...] = jnp.minimum(te, E - 1).astype(jnp.int32)


def _router(flat, gate_w, alpha_row):
    return pl.pallas_call(
        _router_body,
        grid=(2, NTR),
        in_specs=[
            pl.BlockSpec((TMR, H), lambda p, t: (t, 0)),
            pl.BlockSpec((H, E), lambda p, t: (0, 0)),
            pl.BlockSpec((1, E), lambda p, t: (0, 0)),
        ],
        out_specs=[
            pl.BlockSpec((8, TMR), lambda p, t: (0, t)),
            pl.BlockSpec((8, TMR), lambda p, t: (0, t)),
            pl.BlockSpec((1, TE_LANES), lambda p, t: (0, 0)),
        ],
        out_shape=[
            jax.ShapeDtypeStruct((8, N), jnp.int32),
            jax.ShapeDtypeStruct((8, N), jnp.float32),
            jax.ShapeDtypeStruct((1, TE_LANES), jnp.int32),
        ],
        scratch_shapes=[
            pltpu.VMEM((1, E), jnp.float32),
            pltpu.VMEM((1, E), jnp.float32),
        ],
    )(flat, gate_w, alpha_row)


@functools.lru_cache(maxsize=None)
def _make_sc_dispatch():
    mesh = plsc.VectorSubcoreMesh(core_axis_name="c", subcore_axis_name="s",
                                  num_cores=_NC)

    @functools.partial(
        pl.kernel,
        mesh=mesh,
        out_type=jax.ShapeDtypeStruct((P, H), jnp.float32),
        scratch_types=[
            pltpu.VMEM((_TPW // 2, H), jnp.float32),
            pltpu.VMEM((_TPW // 2, H), jnp.float32),
            pltpu.VMEM((_TPW // 2,), jnp.int32),
            pltpu.VMEM((_TPW // 2,), jnp.int32),
            pltpu.VMEM((_TPW // 2,), jnp.int32),
            pltpu.VMEM((_TPW // 2,), jnp.int32),
            pltpu.SemaphoreType.DMA,
            pltpu.SemaphoreType.DMA,
            pltpu.SemaphoreType.DMA,
        ],
    )
    def dispatch_k(x_hbm, dest8_hbm, xg_hbm, rows_a, rows_b,
                   i0a, i0b, i1a, i1b, sr, s0, s1):
        wid = lax.axis_index("s") * _NC + lax.axis_index("c")
        base = wid * _TPW
        hw = _TPW // 2
        ci0 = pltpu.async_copy(dest8_hbm.at[0, pl.ds(base, hw)], i0a, s0)
        ci1 = pltpu.async_copy(dest8_hbm.at[1, pl.ds(base, hw)], i1a, s1)
        ci2 = pltpu.async_copy(dest8_hbm.at[0, pl.ds(base + hw, hw)], i0b, s0)
        ci3 = pltpu.async_copy(dest8_hbm.at[1, pl.ds(base + hw, hw)], i1b, s1)
        ra = pltpu.async_copy(x_hbm.at[pl.ds(base, hw)], rows_a, sr)
        rb = pltpu.async_copy(x_hbm.at[pl.ds(base + hw, hw)], rows_b, sr)
        ci0.wait()
        ci1.wait()
        ci2.wait()
        ci3.wait()
        ra.wait()
        c0 = pltpu.async_copy(rows_a, xg_hbm.at[i0a], s0)
        c1 = pltpu.async_copy(rows_a, xg_hbm.at[i1a], s1)
        rb.wait()
        c2 = pltpu.async_copy(rows_b, xg_hbm.at[i0b], s0)
        c3 = pltpu.async_copy(rows_b, xg_hbm.at[i1b], s1)
        c0.wait()
        c1.wait()
        c2.wait()
        c3.wait()

    return dispatch_k


@functools.lru_cache(maxsize=None)
def _make_sc_combine():
    mesh = plsc.VectorSubcoreMesh(core_axis_name="c", subcore_axis_name="s",
                                  num_cores=_NC)

    @functools.partial(
        pl.kernel,
        mesh=mesh,
        out_type=jax.ShapeDtypeStruct((TOP_K * N, H), jnp.float32),
        scratch_types=(
            [pltpu.VMEM((_TPW // 2, H), jnp.float32) for _ in range(3)]
            + [pltpu.VMEM((_TPW // 2,), jnp.int32) for _ in range(4)]
            + [pltpu.SemaphoreType.DMA, pltpu.SemaphoreType.DMA,
               pltpu.SemaphoreType.DMA]
        ),
    )
    def combine_k(ys_hbm, dest8_hbm, g2_hbm,
                  b0, b1, b2, i0, i1, i2, i3, si, sg, sw):
        wid = lax.axis_index("s") * _NC + lax.axis_index("c")
        base = wid * _TPW
        hw = _TPW // 2
        bufs = [b0, b1, b2]
        idxs = [i0, i1, i2, i3]

        def dst(c):
            slot, half = divmod(c, 2)
            return g2_hbm.at[pl.ds(slot * N + base + half * hw, hw)]

        ics = [pltpu.async_copy(
            dest8_hbm.at[c // 2, pl.ds(base + (c % 2) * hw, hw)],
            idxs[c], si) for c in range(4)]
        ics[0].wait()
        g0 = pltpu.async_copy(ys_hbm.at[idxs[0]], b0, sg)
        ics[1].wait()
        g1 = pltpu.async_copy(ys_hbm.at[idxs[1]], b1, sg)
        ics[2].wait()
        g2c = pltpu.async_copy(ys_hbm.at[idxs[2]], b2, sg)
        g0.wait()
        w0 = pltpu.async_copy(b0, dst(0), sw)
        g1.wait()
        w1 = pltpu.async_copy(b1, dst(1), sw)
        w0.wait()
        ics[3].wait()
        g3 = pltpu.async_copy(ys_hbm.at[idxs[3]], b0, sg)
        g2c.wait()
        w2 = pltpu.async_copy(b2, dst(2), sw)
        g3.wait()
        w3 = pltpu.async_copy(b0, dst(3), sw)
        w1.wait()
        w2.wait()
        w3.wait()

    return combine_k


def _ffn_body(te_ref, xg_ref, f1w_ref, f1b_ref, f2w_ref, f2b_ref, ys_ref):
    xb = xg_ref[...].astype(jnp.bfloat16)
    h1 = jnp.dot(xb, f1w_ref[0], preferred_element_type=jnp.float32)
    h1 = h1 + f1b_ref[0, 0, :][None, :]
    g = 0.5 * h1 * (1.0 + jax.lax.erf(h1 * 0.7071067811865476))
    y = jnp.dot(g.astype(jnp.bfloat16), f2w_ref[0],
                preferred_element_type=jnp.float32)
    ys_ref[...] = y + f2b_ref[0, 0, :][None, :]


def _ffn(xg, f1w, f1b, f2w, f2b, tile_expert):
    grid_spec = pltpu.PrefetchScalarGridSpec(
        num_scalar_prefetch=1,
        grid=(G,),
        in_specs=[
            pl.BlockSpec((TMS, H), lambda g, te: (g, 0)),
            pl.BlockSpec((1, H, I), lambda g, te: (te[g], 0, 0)),
            pl.BlockSpec((1, 1, I), lambda g, te: (te[g], 0, 0)),
            pl.BlockSpec((1, I, H), lambda g, te: (te[g], 0, 0)),
            pl.BlockSpec((1, 1, H), lambda g, te: (te[g], 0, 0)),
        ],
        out_specs=pl.BlockSpec((TMS, H), lambda g, te: (g, 0)),
    )
    return pl.pallas_call(
        _ffn_body,
        grid_spec=grid_spec,
        out_shape=jax.ShapeDtypeStruct((P, H), jnp.float32),
    )(tile_expert, xg, f1w, f1b, f2w, f2b)


def _wadd_body(g_ref, w_ref, out_ref):
    w0 = w_ref[0, :][:, None]
    w1 = w_ref[1, :][:, None]
    out_ref[...] = g_ref[0] * w0 + g_ref[1] * w1


def _combine_add(g2, w8):
    tm = 512
    return pl.pallas_call(
        _wadd_body,
        grid=(N // tm,),
        in_specs=[
            pl.BlockSpec((2, tm, H), lambda t: (0, t, 0)),
            pl.BlockSpec((8, tm), lambda t: (0, t)),
        ],
        out_specs=pl.BlockSpec((tm, H), lambda t: (t, 0)),
        out_shape=jax.ShapeDtypeStruct((N, H), jnp.float32),
    )(g2, w8)


def _sc_dispatch(x, dest8):
    return _make_sc_dispatch()(x, dest8)


def _sc_combine(ys, dest8):
    return _make_sc_combine()(ys, dest8)


@jax.jit
def _moe(flat, gate_w, alpha_row, f1w, f1b, f2w, f2b):
    dest8, w8, te = _router(flat, gate_w, alpha_row)
    tile_expert = te[0, :G]
    xg = _sc_dispatch(flat, dest8)
    ys = _ffn(xg, f1w, f1b, f2w, f2b, tile_expert)
    g2 = _sc_combine(ys, dest8)
    return _combine_add(g2.reshape(TOP_K, N, H), w8)


def kernel(hidden_states, gate_w, fc1_w, fc1_b, fc2_w, fc2_b, alpha):
    b, s, h = hidden_states.shape
    flat = hidden_states.reshape(-1, h)
    f1w = fc1_w.astype(jnp.bfloat16)
    f2w = fc2_w.astype(jnp.bfloat16)
    f1b = fc1_b.reshape(E, 1, I)
    f2b = fc2_b.reshape(E, 1, H)
    out = _moe(flat, gate_w, alpha.reshape(1, E), f1w, f1b, f2w, f2b)
    return out.reshape(b, s, h)


# speedup vs baseline: 3.5122x; 1.0153x over previous
import functools

import jax
import jax.numpy as jnp
from jax import lax
from jax.experimental import pallas as pl
from jax.experimental.pallas import tpu as pltpu
from jax.experimental.pallas import tpu_sc as plsc

E = 8
TOP_K = 2
H = 1024
I = 1024
N = 2048
TMS = 256
P = N * TOP_K + E * TMS
G = P // TMS
TE_LANES = 128

TMR = 512
NTR = N // TMR

_NC = 2
_NS = 16
_NW = _NC * _NS
_TPW = N // _NW


def _router_body(x_ref, gw_ref, alpha_ref, dest8_ref, w8_ref, te_ref,
                 xbi_ref, cnt_ref, run_ref):
    p = pl.program_id(0)
    t = pl.program_id(1)
    x = x_ref[...]
    lo = jax.lax.bitcast_convert_type(
        x[:, :H // 2].astype(jnp.bfloat16), jnp.uint16).astype(jnp.uint32)
    hi = jax.lax.bitcast_convert_type(
        x[:, H // 2:].astype(jnp.bfloat16), jnp.uint16).astype(jnp.uint32)
    xbi_ref[...] = jax.lax.bitcast_convert_type(
        lo | (hi << 16), jnp.int32)
    logits = jnp.dot(x, gw_ref[...], preferred_element_type=jnp.float32)
    probs = jax.nn.softmax(logits, axis=-1)
    m1 = jnp.max(probs, axis=-1, keepdims=True)
    masked = jnp.where(probs >= m1, -1.0, probs)
    m2 = jnp.max(masked, axis=-1, keepdims=True)
    sel = (probs >= m2).astype(jnp.float32)
    colsum = jnp.sum(sel, axis=0, keepdims=True)

    @pl.when(p == 0)
    def _():
        prev = jnp.where(t == 0, jnp.zeros_like(colsum), cnt_ref[...])
        cnt_ref[...] = prev + colsum

    @pl.when(p == 1)
    def _():
        cnt = cnt_ref[...]
        padded = jnp.floor((cnt + (TMS - 1)) * (1.0 / TMS)) * TMS
        triu = (jax.lax.broadcasted_iota(jnp.int32, (E, E), 0)
                <= jax.lax.broadcasted_iota(jnp.int32, (E, E), 1)
                ).astype(jnp.float32)
        cum = jnp.dot(padded, triu, preferred_element_type=jnp.float32)
        offs = cum - padded

        run = jnp.where(t == 0, jnp.zeros_like(colsum), run_ref[...])
        run_ref[...] = run + colsum
        tril = (jax.lax.broadcasted_iota(jnp.int32, (TMR, TMR), 1)
                < jax.lax.broadcasted_iota(jnp.int32, (TMR, TMR), 0)
                ).astype(jnp.float32)
        ranks = jnp.dot(tril, sel, preferred_element_type=jnp.float32)
        dest_all = offs + run + ranks

        lane = jax.lax.broadcasted_iota(jnp.int32, probs.shape, 1)
        i1 = jnp.min(jnp.where(probs >= m1, lane, E), axis=-1, keepdims=True)
        i2 = jnp.min(jnp.where((probs >= m2) & (lane != i1), lane, E),
                     axis=-1, keepdims=True)
        hit1 = lane == i1
        hit2 = lane == i2
        d0 = jnp.sum(jnp.where(hit1, dest_all, 0.0), axis=-1)
        d1 = jnp.sum(jnp.where(hit2, dest_all, 0.0), axis=-1)
        wa = probs * alpha_ref[...]
        w0 = jnp.sum(jnp.where(hit1, wa, 0.0), axis=-1)
        w1 = jnp.sum(jnp.where(hit2, wa, 0.0), axis=-1)

        su = jax.lax.broadcasted_iota(jnp.int32, (8, TMR), 0)
        dest8_ref[...] = jnp.where(
            su == 0, d0[None, :], jnp.where(su == 1, d1[None, :], 0.0)
        ).astype(jnp.int32)
        w8_ref[...] = jnp.where(
            su == 0, w0[None, :], jnp.where(su == 1, w1[None, :], 0.0))

        gt = (jax.lax.broadcasted_iota(jnp.int32, (1, TE_LANES), 1)
              * TMS).astype(jnp.float32)
        te = jnp.zeros((1, TE_LANES), jnp.float32)
        for e in range(E):
            te = te + (gt >= cum[0, e]).astype(jnp.float32)
        te_ref[...] = jnp.minimum(te, E - 1).astype(jnp.int32)


def _router(flat, gate_w, alpha_row):
    return pl.pallas_call(
        _router_body,
        grid=(2, NTR),
        in_specs=[
            pl.BlockSpec((TMR, H), lambda p, t: (t, 0)),
            pl.BlockSpec((H, E), lambda p, t: (0, 0)),
            pl.BlockSpec((1, E), lambda p, t: (0, 0)),
        ],
        out_specs=[
            pl.BlockSpec((8, TMR), lambda p, t: (0, t)),
            pl.BlockSpec((8, TMR), lambda p, t: (0, t)),
            pl.BlockSpec((1, TE_LANES), lambda p, t: (0, 0)),
            pl.BlockSpec((TMR, H // 2), lambda p, t: (t, 0)),
        ],
        out_shape=[
            jax.ShapeDtypeStruct((8, N), jnp.int32),
            jax.ShapeDtypeStruct((8, N), jnp.float32),
            jax.ShapeDtypeStruct((1, TE_LANES), jnp.int32),
            jax.ShapeDtypeStruct((N, H // 2), jnp.int32),
        ],
        scratch_shapes=[
            pltpu.VMEM((1, E), jnp.float32),
            pltpu.VMEM((1, E), jnp.float32),
        ],
    )(flat, gate_w, alpha_row)


@functools.lru_cache(maxsize=None)
def _make_sc_dispatch():
    mesh = plsc.VectorSubcoreMesh(core_axis_name="c", subcore_axis_name="s",
                                  num_cores=_NC)

    @functools.partial(
        pl.kernel,
        mesh=mesh,
        out_type=jax.ShapeDtypeStruct((P, H // 2), jnp.int32),
        scratch_types=[
            pltpu.VMEM((_TPW // 2, H // 2), jnp.int32),
            pltpu.VMEM((_TPW // 2, H // 2), jnp.int32),
            pltpu.VMEM((_TPW // 2,), jnp.int32),
            pltpu.VMEM((_TPW // 2,), jnp.int32),
            pltpu.VMEM((_TPW // 2,), jnp.int32),
            pltpu.VMEM((_TPW // 2,), jnp.int32),
            pltpu.SemaphoreType.DMA,
            pltpu.SemaphoreType.DMA,
            pltpu.SemaphoreType.DMA,
        ],
    )
    def dispatch_k(x_hbm, dest8_hbm, xg_hbm, rows_a, rows_b,
                   i0a, i0b, i1a, i1b, sr, s0, s1):
        wid = lax.axis_index("s") * _NC + lax.axis_index("c")
        base = wid * _TPW
        hw = _TPW // 2
        ci0 = pltpu.async_copy(dest8_hbm.at[0, pl.ds(base, hw)], i0a, s0)
        ci1 = pltpu.async_copy(dest8_hbm.at[1, pl.ds(base, hw)], i1a, s1)
        ci2 = pltpu.async_copy(dest8_hbm.at[0, pl.ds(base + hw, hw)], i0b, s0)
        ci3 = pltpu.async_copy(dest8_hbm.at[1, pl.ds(base + hw, hw)], i1b, s1)
        ra = pltpu.async_copy(x_hbm.at[pl.ds(base, hw)], rows_a, sr)
        rb = pltpu.async_copy(x_hbm.at[pl.ds(base + hw, hw)], rows_b, sr)
        ci0.wait()
        ci1.wait()
        ci2.wait()
        ci3.wait()
        ra.wait()
        c0 = pltpu.async_copy(rows_a, xg_hbm.at[i0a], s0)
        c1 = pltpu.async_copy(rows_a, xg_hbm.at[i1a], s1)
        rb.wait()
        c2 = pltpu.async_copy(rows_b, xg_hbm.at[i0b], s0)
        c3 = pltpu.async_copy(rows_b, xg_hbm.at[i1b], s1)
        c0.wait()
        c1.wait()
        c2.wait()
        c3.wait()

    return dispatch_k


@functools.lru_cache(maxsize=None)
def _make_sc_combine():
    mesh = plsc.VectorSubcoreMesh(core_axis_name="c", subcore_axis_name="s",
                                  num_cores=_NC)

    @functools.partial(
        pl.kernel,
        mesh=mesh,
        out_type=jax.ShapeDtypeStruct((TOP_K * N, H), jnp.float32),
        scratch_types=(
            [pltpu.VMEM((_TPW // 2, H), jnp.float32) for _ in range(3)]
            + [pltpu.VMEM((_TPW // 2,), jnp.int32) for _ in range(4)]
            + [pltpu.SemaphoreType.DMA, pltpu.SemaphoreType.DMA,
               pltpu.SemaphoreType.DMA]
        ),
    )
    def combine_k(ys_hbm, dest8_hbm, g2_hbm,
                  b0, b1, b2, i0, i1, i2, i3, si, sg, sw):
        wid = lax.axis_index("s") * _NC + lax.axis_index("c")
        base = wid * _TPW
        hw = _TPW // 2
        bufs = [b0, b1, b2]
        idxs = [i0, i1, i2, i3]

        def dst(c):
            slot, half = divmod(c, 2)
            return g2_hbm.at[pl.ds(slot * N + base + half * hw, hw)]

        ics = [pltpu.async_copy(
            dest8_hbm.at[c // 2, pl.ds(base + (c % 2) * hw, hw)],
            idxs[c], si) for c in range(4)]
        ics[0].wait()
        g0 = pltpu.async_copy(ys_hbm.at[idxs[0]], b0, sg)
        ics[1].wait()
        g1 = pltpu.async_copy(ys_hbm.at[idxs[1]], b1, sg)
        ics[2].wait()
        g2c = pltpu.async_copy(ys_hbm.at[idxs[2]], b2, sg)
        g0.wait()
        w0 = pltpu.async_copy(b0, dst(0), sw)
        g1.wait()
        w1 = pltpu.async_copy(b1, dst(1), sw)
        w0.wait()
        ics[3].wait()
        g3 = pltpu.async_copy(ys_hbm.at[idxs[3]], b0, sg)
        g2c.wait()
        w2 = pltpu.async_copy(b2, dst(2), sw)
        g3.wait()
        w3 = pltpu.async_copy(b0, dst(3), sw)
        w1.wait()
        w2.wait()
        w3.wait()

    return combine_k


def _ffn_body(te_ref, xg_ref, f1wl_ref, f1wh_ref, f1b_ref, f2w_ref, f2b_ref,
              ys_ref):
    xi = jax.lax.bitcast_convert_type(xg_ref[...], jnp.uint32)
    xlo = jax.lax.bitcast_convert_type(
        (xi & 0xFFFF).astype(jnp.uint16), jnp.bfloat16)
    xhi = jax.lax.bitcast_convert_type(
        (xi >> 16).astype(jnp.uint16), jnp.bfloat16)
    h1 = (jnp.dot(xlo, f1wl_ref[0], preferred_element_type=jnp.float32)
          + jnp.dot(xhi, f1wh_ref[0], preferred_element_type=jnp.float32))
    h1 = h1 + f1b_ref[0, 0, :][None, :]
    g = 0.5 * h1 * (1.0 + jax.lax.erf(h1 * 0.7071067811865476))
    y = jnp.dot(g.astype(jnp.bfloat16), f2w_ref[0],
                preferred_element_type=jnp.float32)
    ys_ref[...] = y + f2b_ref[0, 0, :][None, :]


def _ffn(xg, f1wl, f1wh, f1b, f2w, f2b, tile_expert):
    grid_spec = pltpu.PrefetchScalarGridSpec(
        num_scalar_prefetch=1,
        grid=(G,),
        in_specs=[
            pl.BlockSpec((TMS, H // 2), lambda g, te: (g, 0)),
            pl.BlockSpec((1, H // 2, I), lambda g, te: (te[g], 0, 0)),
            pl.BlockSpec((1, H // 2, I), lambda g, te: (te[g], 0, 0)),
            pl.BlockSpec((1, 1, I), lambda g, te: (te[g], 0, 0)),
            pl.BlockSpec((1, I, H), lambda g, te: (te[g], 0, 0)),
            pl.BlockSpec((1, 1, H), lambda g, te: (te[g], 0, 0)),
        ],
        out_specs=pl.BlockSpec((TMS, H), lambda g, te: (g, 0)),
    )
    return pl.pallas_call(
        _ffn_body,
        grid_spec=grid_spec,
        out_shape=jax.ShapeDtypeStruct((P, H), jnp.float32),
    )(tile_expert, xg, f1wl, f1wh, f1b, f2w, f2b)


def _wadd_body(g_ref, w_ref, out_ref):
    w0 = w_ref[0, :][:, None]
    w1 = w_ref[1, :][:, None]
    out_ref[...] = g_ref[0] * w0 + g_ref[1] * w1


def _combine_add(g2, w8):
    tm = 512
    return pl.pallas_call(
        _wadd_body,
        grid=(N // tm,),
        in_specs=[
            pl.BlockSpec((2, tm, H), lambda t: (0, t, 0)),
            pl.BlockSpec((8, tm), lambda t: (0, t)),
        ],
        out_specs=pl.BlockSpec((tm, H), lambda t: (t, 0)),
        out_shape=jax.ShapeDtypeStruct((N, H), jnp.float32),
    )(g2, w8)


def _sc_dispatch(x, dest8):
    return _make_sc_dispatch()(x, dest8)


def _sc_combine(ys, dest8):
    return _make_sc_combine()(ys, dest8)


@jax.jit
def _moe(flat, gate_w, alpha_row, f1wl, f1wh, f1b, f2w, f2b):
    dest8, w8, te, xbi = _router(flat, gate_w, alpha_row)
    tile_expert = te[0, :G]
    xg = _sc_dispatch(xbi, dest8)
    ys = _ffn(xg, f1wl, f1wh, f1b, f2w, f2b, tile_expert)
    g2 = _sc_combine(ys, dest8)
    return _combine_add(g2.reshape(TOP_K, N, H), w8)


def kernel(hidden_states, gate_w, fc1_w, fc1_b, fc2_w, fc2_b, alpha):
    b, s, h = hidden_states.shape
    flat = hidden_states.reshape(-1, h)
    f1wl = fc1_w[:, :H // 2, :].astype(jnp.bfloat16)
    f1wh = fc1_w[:, H // 2:, :].astype(jnp.bfloat16)
    f2w = fc2_w.astype(jnp.bfloat16)
    f1b = fc1_b.reshape(E, 1, I)
    f2b = fc2_b.reshape(E, 1, H)
    out = _moe(flat, gate_w, alpha.reshape(1, E), f1wl, f1wh, f1b, f2w, f2b)
    return out.reshape(b, s, h)


# speedup vs baseline: 3.7393x; 1.0646x over previous
import functools

import jax
import jax.numpy as jnp
from jax import lax
from jax.experimental import pallas as pl
from jax.experimental.pallas import tpu as pltpu
from jax.experimental.pallas import tpu_sc as plsc

E = 8
TOP_K = 2
H = 1024
I = 1024
N = 2048
TMS = 256
P = N * TOP_K + E * TMS
G = P // TMS
TE_LANES = 128

TMR = 512
NTR = N // TMR

_NC = 2
_NS = 16
_NW = _NC * _NS
_TPW = N // _NW


def _router_body(x_ref, gw_ref, alpha_ref, dest8_ref, w8_ref, te_ref,
                 xbi_ref, cnt_ref, run_ref):
    p = pl.program_id(0)
    t = pl.program_id(1)
    x = x_ref[...]
    lo = jax.lax.bitcast_convert_type(
        x[:, :H // 2].astype(jnp.bfloat16), jnp.uint16).astype(jnp.uint32)
    hi = jax.lax.bitcast_convert_type(
        x[:, H // 2:].astype(jnp.bfloat16), jnp.uint16).astype(jnp.uint32)
    xbi_ref[...] = jax.lax.bitcast_convert_type(
        lo | (hi << 16), jnp.int32)
    logits = jnp.dot(x, gw_ref[...], preferred_element_type=jnp.float32)
    probs = jax.nn.softmax(logits, axis=-1)
    m1 = jnp.max(probs, axis=-1, keepdims=True)
    masked = jnp.where(probs >= m1, -1.0, probs)
    m2 = jnp.max(masked, axis=-1, keepdims=True)
    sel = (probs >= m2).astype(jnp.float32)
    colsum = jnp.sum(sel, axis=0, keepdims=True)

    @pl.when(p == 0)
    def _():
        prev = jnp.where(t == 0, jnp.zeros_like(colsum), cnt_ref[...])
        cnt_ref[...] = prev + colsum

    @pl.when(p == 1)
    def _():
        cnt = cnt_ref[...]
        padded = jnp.floor((cnt + (TMS - 1)) * (1.0 / TMS)) * TMS
        triu = (jax.lax.broadcasted_iota(jnp.int32, (E, E), 0)
                <= jax.lax.broadcasted_iota(jnp.int32, (E, E), 1)
                ).astype(jnp.float32)
        cum = jnp.dot(padded, triu, preferred_element_type=jnp.float32)
        offs = cum - padded

        run = jnp.where(t == 0, jnp.zeros_like(colsum), run_ref[...])
        run_ref[...] = run + colsum
        tril = (jax.lax.broadcasted_iota(jnp.int32, (TMR, TMR), 1)
                < jax.lax.broadcasted_iota(jnp.int32, (TMR, TMR), 0)
                ).astype(jnp.float32)
        ranks = jnp.dot(tril, sel, preferred_element_type=jnp.float32)
        dest_all = offs + run + ranks

        lane = jax.lax.broadcasted_iota(jnp.int32, probs.shape, 1)
        i1 = jnp.min(jnp.where(probs >= m1, lane, E), axis=-1, keepdims=True)
        i2 = jnp.min(jnp.where((probs >= m2) & (lane != i1), lane, E),
                     axis=-1, keepdims=True)
        hit1 = lane == i1
        hit2 = lane == i2
        d0 = jnp.sum(jnp.where(hit1, dest_all, 0.0), axis=-1)
        d1 = jnp.sum(jnp.where(hit2, dest_all, 0.0), axis=-1)
        wa = probs * alpha_ref[...]
        w0 = jnp.sum(jnp.where(hit1, wa, 0.0), axis=-1)
        w1 = jnp.sum(jnp.where(hit2, wa, 0.0), axis=-1)

        su = jax.lax.broadcasted_iota(jnp.int32, (8, TMR), 0)
        dest8_ref[...] = jnp.where(
            su == 0, d0[None, :], jnp.where(su == 1, d1[None, :], 0.0)
        ).astype(jnp.int32)
        w8_ref[...] = jnp.where(
            su == 0, w0[None, :], jnp.where(su == 1, w1[None, :], 0.0))

        gt = (jax.lax.broadcasted_iota(jnp.int32, (1, TE_LANES), 1)
              * TMS).astype(jnp.float32)
        te = jnp.zeros((1, TE_LANES), jnp.float32)
        for e in range(E):
            te = te + (gt >= cum[0, e]).astype(jnp.float32)
        te_ref[...] = jnp.minimum(te, E - 1).astype(jnp.int32)


def _router(flat, gate_w, alpha_row):
    return pl.pallas_call(
        _router_body,
        grid=(2, NTR),
        in_specs=[
            pl.BlockSpec((TMR, H), lambda p, t: (t, 0)),
            pl.BlockSpec((H, E), lambda p, t: (0, 0)),
            pl.BlockSpec((1, E), lambda p, t: (0, 0)),
        ],
        out_specs=[
            pl.BlockSpec((8, TMR), lambda p, t: (0, t)),
            pl.BlockSpec((8, TMR), lambda p, t: (0, t)),
            pl.BlockSpec((1, TE_LANES), lambda p, t: (0, 0)),
            pl.BlockSpec((TMR, H // 2), lambda p, t: (t, 0)),
        ],
        out_shape=[
            jax.ShapeDtypeStruct((8, N), jnp.int32),
            jax.ShapeDtypeStruct((8, N), jnp.float32),
            jax.ShapeDtypeStruct((1, TE_LANES), jnp.int32),
            jax.ShapeDtypeStruct((N, H // 2), jnp.int32),
        ],
        scratch_shapes=[
            pltpu.VMEM((1, E), jnp.float32),
            pltpu.VMEM((1, E), jnp.float32),
        ],
    )(flat, gate_w, alpha_row)


@functools.lru_cache(maxsize=None)
def _make_sc_dispatch():
    mesh = plsc.VectorSubcoreMesh(core_axis_name="c", subcore_axis_name="s",
                                  num_cores=_NC)

    @functools.partial(
        pl.kernel,
        mesh=mesh,
        out_type=jax.ShapeDtypeStruct((P, H // 2), jnp.int32),
        scratch_types=[
            pltpu.VMEM((_TPW // 2, H // 2), jnp.int32),
            pltpu.VMEM((_TPW // 2, H // 2), jnp.int32),
            pltpu.VMEM((_TPW // 2,), jnp.int32),
            pltpu.VMEM((_TPW // 2,), jnp.int32),
            pltpu.VMEM((_TPW // 2,), jnp.int32),
            pltpu.VMEM((_TPW // 2,), jnp.int32),
            pltpu.SemaphoreType.DMA,
            pltpu.SemaphoreType.DMA,
            pltpu.SemaphoreType.DMA,
        ],
    )
    def dispatch_k(x_hbm, dest8_hbm, xg_hbm, rows_a, rows_b,
                   i0a, i0b, i1a, i1b, sr, s0, s1):
        wid = lax.axis_index("s") * _NC + lax.axis_index("c")
        base = wid * _TPW
        hw = _TPW // 2
        ci0 = pltpu.async_copy(dest8_hbm.at[0, pl.ds(base, hw)], i0a, s0)
        ci1 = pltpu.async_copy(dest8_hbm.at[1, pl.ds(base, hw)], i1a, s1)
        ci2 = pltpu.async_copy(dest8_hbm.at[0, pl.ds(base + hw, hw)], i0b, s0)
        ci3 = pltpu.async_copy(dest8_hbm.at[1, pl.ds(base + hw, hw)], i1b, s1)
        ra = pltpu.async_copy(x_hbm.at[pl.ds(base, hw)], rows_a, sr)
        rb = pltpu.async_copy(x_hbm.at[pl.ds(base + hw, hw)], rows_b, sr)
        ci0.wait()
        ci1.wait()
        ci2.wait()
        ci3.wait()
        ra.wait()
        c0 = pltpu.async_copy(rows_a, xg_hbm.at[i0a], s0)
        c1 = pltpu.async_copy(rows_a, xg_hbm.at[i1a], s1)
        rb.wait()
        c2 = pltpu.async_copy(rows_b, xg_hbm.at[i0b], s0)
        c3 = pltpu.async_copy(rows_b, xg_hbm.at[i1b], s1)
        c0.wait()
        c1.wait()
        c2.wait()
        c3.wait()

    return dispatch_k


@functools.lru_cache(maxsize=None)
def _make_sc_combine():
    mesh = plsc.VectorSubcoreMesh(core_axis_name="c", subcore_axis_name="s",
                                  num_cores=_NC)

    @functools.partial(
        pl.kernel,
        mesh=mesh,
        out_type=jax.ShapeDtypeStruct((TOP_K * N, H // 2), jnp.int32),
        scratch_types=(
            [pltpu.VMEM((_TPW // 2, H // 2), jnp.int32) for _ in range(3)]
            + [pltpu.VMEM((_TPW // 2,), jnp.int32) for _ in range(4)]
            + [pltpu.SemaphoreType.DMA, pltpu.SemaphoreType.DMA,
               pltpu.SemaphoreType.DMA]
        ),
    )
    def combine_k(ys_hbm, dest8_hbm, g2_hbm,
                  b0, b1, b2, i0, i1, i2, i3, si, sg, sw):
        wid = lax.axis_index("s") * _NC + lax.axis_index("c")
        base = wid * _TPW
        hw = _TPW // 2
        bufs = [b0, b1, b2]
        idxs = [i0, i1, i2, i3]

        def dst(c):
            slot, half = divmod(c, 2)
            return g2_hbm.at[pl.ds(slot * N + base + half * hw, hw)]

        ics = [pltpu.async_copy(
            dest8_hbm.at[c // 2, pl.ds(base + (c % 2) * hw, hw)],
            idxs[c], si) for c in range(4)]
        ics[0].wait()
        g0 = pltpu.async_copy(ys_hbm.at[idxs[0]], b0, sg)
        ics[1].wait()
        g1 = pltpu.async_copy(ys_hbm.at[idxs[1]], b1, sg)
        ics[2].wait()
        g2c = pltpu.async_copy(ys_hbm.at[idxs[2]], b2, sg)
        g0.wait()
        w0 = pltpu.async_copy(b0, dst(0), sw)
        g1.wait()
        w1 = pltpu.async_copy(b1, dst(1), sw)
        w0.wait()
        ics[3].wait()
        g3 = pltpu.async_copy(ys_hbm.at[idxs[3]], b0, sg)
        g2c.wait()
        w2 = pltpu.async_copy(b2, dst(2), sw)
        g3.wait()
        w3 = pltpu.async_copy(b0, dst(3), sw)
        w1.wait()
        w2.wait()
        w3.wait()

    return combine_k


def _ffn_body(te_ref, xg_ref, f1wl_ref, f1wh_ref, f1b_ref, f2w_ref, f2b_ref,
              ys_ref):
    xi = jax.lax.bitcast_convert_type(xg_ref[...], jnp.uint32)
    xlo = jax.lax.bitcast_convert_type(
        (xi & 0xFFFF).astype(jnp.uint16), jnp.bfloat16)
    xhi = jax.lax.bitcast_convert_type(
        (xi >> 16).astype(jnp.uint16), jnp.bfloat16)
    h1 = (jnp.dot(xlo, f1wl_ref[0], preferred_element_type=jnp.float32)
          + jnp.dot(xhi, f1wh_ref[0], preferred_element_type=jnp.float32))
    h1 = h1 + f1b_ref[0, 0, :][None, :]
    g = 0.5 * h1 * (1.0 + jax.lax.erf(h1 * 0.7071067811865476))
    y = jnp.dot(g.astype(jnp.bfloat16), f2w_ref[0],
                preferred_element_type=jnp.float32)
    y = y + f2b_ref[0, 0, :][None, :]
    ylo = jax.lax.bitcast_convert_type(
        y[:, :H // 2].astype(jnp.bfloat16), jnp.uint16).astype(jnp.uint32)
    yhi = jax.lax.bitcast_convert_type(
        y[:, H // 2:].astype(jnp.bfloat16), jnp.uint16).astype(jnp.uint32)
    ys_ref[...] = jax.lax.bitcast_convert_type(ylo | (yhi << 16), jnp.int32)


def _ffn(xg, f1wl, f1wh, f1b, f2w, f2b, tile_expert):
    grid_spec = pltpu.PrefetchScalarGridSpec(
        num_scalar_prefetch=1,
        grid=(G,),
        in_specs=[
            pl.BlockSpec((TMS, H // 2), lambda g, te: (g, 0)),
            pl.BlockSpec((1, H // 2, I), lambda g, te: (te[g], 0, 0)),
            pl.BlockSpec((1, H // 2, I), lambda g, te: (te[g], 0, 0)),
            pl.BlockSpec((1, 1, I), lambda g, te: (te[g], 0, 0)),
            pl.BlockSpec((1, I, H), lambda g, te: (te[g], 0, 0)),
            pl.BlockSpec((1, 1, H), lambda g, te: (te[g], 0, 0)),
        ],
        out_specs=pl.BlockSpec((TMS, H // 2), lambda g, te: (g, 0)),
    )
    return pl.pallas_call(
        _ffn_body,
        grid_spec=grid_spec,
        out_shape=jax.ShapeDtypeStruct((P, H // 2), jnp.int32),
    )(tile_expert, xg, f1wl, f1wh, f1b, f2w, f2b)


def _unpack16(xi32):
    xu = jax.lax.bitcast_convert_type(xi32, jnp.uint32)
    lo = jax.lax.bitcast_convert_type(
        (xu & 0xFFFF).astype(jnp.uint16), jnp.bfloat16).astype(jnp.float32)
    hi = jax.lax.bitcast_convert_type(
        (xu >> 16).astype(jnp.uint16), jnp.bfloat16).astype(jnp.float32)
    return lo, hi


def _wadd_body(g_ref, w_ref, out_ref):
    w0 = w_ref[0, :][:, None]
    w1 = w_ref[1, :][:, None]
    lo0, hi0 = _unpack16(g_ref[0])
    lo1, hi1 = _unpack16(g_ref[1])
    out_ref[:, :H // 2] = lo0 * w0 + lo1 * w1
    out_ref[:, H // 2:] = hi0 * w0 + hi1 * w1


def _combine_add(g2, w8):
    tm = 512
    return pl.pallas_call(
        _wadd_body,
        grid=(N // tm,),
        in_specs=[
            pl.BlockSpec((2, tm, H // 2), lambda t: (0, t, 0)),
            pl.BlockSpec((8, tm), lambda t: (0, t)),
        ],
        out_specs=pl.BlockSpec((tm, H), lambda t: (t, 0)),
        out_shape=jax.ShapeDtypeStruct((N, H), jnp.float32),
    )(g2, w8)


def _sc_dispatch(x, dest8):
    return _make_sc_dispatch()(x, dest8)


def _sc_combine(ys, dest8):
    return _make_sc_combine()(ys, dest8)


@jax.jit
def _moe(flat, gate_w, alpha_row, f1wl, f1wh, f1b, f2w, f2b):
    dest8, w8, te, xbi = _router(flat, gate_w, alpha_row)
    tile_expert = te[0, :G]
    xg = _sc_dispatch(xbi, dest8)
    ys = _ffn(xg, f1wl, f1wh, f1b, f2w, f2b, tile_expert)
    g2 = _sc_combine(ys, dest8)
    return _combine_add(g2.reshape(TOP_K, N, H // 2), w8)


def kernel(hidden_states, gate_w, fc1_w, fc1_b, fc2_w, fc2_b, alpha):
    b, s, h = hidden_states.shape
    flat = hidden_states.reshape(-1, h)
    f1wl = fc1_w[:, :H // 2, :].astype(jnp.bfloat16)
    f1wh = fc1_w[:, H // 2:, :].astype(jnp.bfloat16)
    f2w = fc2_w.astype(jnp.bfloat16)
    f1b = fc1_b.reshape(E, 1, I)
    f2b = fc2_b.reshape(E, 1, H)
    out = _moe(flat, gate_w, alpha.reshape(1, E), f1wl, f1wh, f1b, f2w, f2b)
    return out.reshape(b, s, h)
